# baseline (speedup 1.0000x reference)
import functools

import jax
import jax.numpy as jnp
import numpy as np
from jax import lax
from jax.experimental import pallas as pl
from jax.experimental.pallas import tpu as pltpu
from jax.experimental.pallas import tpu_sc as plsc

STATE_DIM = 2048
ACTION_DIM = 32768
B = 128
TOPK = 8
VBLK = 4096
NB = ACTION_DIM // VBLK
LEAF = 128
NLEAF = ACTION_DIM // LEAF
_HIGHEST = lax.Precision.HIGHEST


def _mlp_head(state, W1, b1, W2, b2, W3, b3):
    e = jnp.maximum(lax.dot_general(state, W1, (((1,), (1,)), ((), ())),
                                    precision=None,
                                    preferred_element_type=jnp.float32) + b1[None, :], 0.0)
    e = jnp.maximum(lax.dot_general(e, W2, (((1,), (1,)), ((), ())),
                                    precision=None,
                                    preferred_element_type=jnp.float32) + b2[None, :], 0.0)
    h = jnp.maximum(lax.dot_general(e, W3, (((1,), (1,)), ((), ())),
                                    precision=None,
                                    preferred_element_type=jnp.float32) + b3[None, :], 0.0)
    return h


def _stage1_body(state_ref, W1_ref, b1_ref, W2_ref, b2_ref, W3_ref, b3_ref,
                 W4_ref, b4_ref, noise_ref,
                 pert_ref, pbmax_ref, m_ref, s_ref,
                 h_s, m_s, s_s):
    i = pl.program_id(0)

    @pl.when(i == 0)
    def _init():
        h_s[...] = _mlp_head(state_ref[...], W1_ref[...], b1_ref[...],
                             W2_ref[...], b2_ref[...], W3_ref[...], b3_ref[...])
        m_s[...] = jnp.full((B, 1), -jnp.inf, jnp.float32)
        s_s[...] = jnp.zeros((B, 1), jnp.float32)

    logits = jnp.maximum(
        lax.dot_general(h_s[...], W4_ref[...], (((1,), (1,)), ((), ())),
                        precision=None,
                        preferred_element_type=jnp.float32) + b4_ref[...][None, :],
        0.0)
    pert = logits + noise_ref[...]
    pert_ref[...] = pert
    pbmax_ref[...] = jnp.max(pert.reshape(B, VBLK // LEAF, LEAF),
                             axis=-1).reshape(1, B, VBLK // LEAF)

    lb = jnp.max(logits, axis=1, keepdims=True)
    m_new = jnp.maximum(m_s[...], lb)
    s_s[...] = (s_s[...] * jnp.exp(m_s[...] - m_new)
                + jnp.sum(jnp.exp(logits - m_new), axis=1, keepdims=True))
    m_s[...] = m_new

    @pl.when(i == NB - 1)
    def _fin():
        m_ref[...] = m_s[...]
        s_ref[...] = s_s[...]


@functools.partial(jax.jit, static_argnames=("interpret",))
def _stage1(state, W1, b1, W2, b2, W3, b3, W4, b4, noise, interpret=False):
    const2 = lambda r: pl.BlockSpec(None, lambda i: tuple([0] * r))
    return pl.pallas_call(
        _stage1_body,
        grid=(NB,),
        in_specs=[
            const2(2), const2(2), const2(1), const2(2), const2(1),
            const2(2), const2(1),
            pl.BlockSpec((VBLK, 64), lambda i: (i, 0)),
            pl.BlockSpec((VBLK,), lambda i: (i,)),
            pl.BlockSpec((B, VBLK), lambda i: (0, i)),
        ],
        out_specs=[
            pl.BlockSpec((B, VBLK), lambda i: (0, i)),
            pl.BlockSpec((1, B, VBLK // LEAF), lambda i: (i, 0, 0)),
            pl.BlockSpec((B, 1), lambda i: (0, 0)),
            pl.BlockSpec((B, 1), lambda i: (0, 0)),
        ],
        out_shape=[
            jax.ShapeDtypeStruct((B, ACTION_DIM), jnp.float32),
            jax.ShapeDtypeStruct((NB, B, VBLK // LEAF), jnp.float32),
            jax.ShapeDtypeStruct((B, 1), jnp.float32),
            jax.ShapeDtypeStruct((B, 1), jnp.float32),
        ],
        scratch_shapes=[
            pltpu.VMEM((B, 64), jnp.float32),
            pltpu.VMEM((B, 1), jnp.float32),
            pltpu.VMEM((B, 1), jnp.float32),
        ],
        interpret=interpret,
    )(state, W1, b1, W2, b2, W3, b3, W4, b4, noise)


_NW = 32
_RPW = B // _NW
_NCAND = TOPK * LEAF
_BIG = 2 ** 30


def _iota16():
    return lax.iota(jnp.int32, 16)


def _vextract(vec, lane, fill):
    return jnp.max(jnp.where(_iota16() == lane, vec, fill))


def _argmax8(load_val, load_idx, nvec, mask_out, record, unroll):
    neg_inf = jnp.float32(-jnp.inf)
    for j in range(TOPK):
        def scan(v, carry):
            cmax, cidx = carry
            x = load_val(v)
            ix = load_idx(v)
            take = x > cmax
            return jnp.where(take, x, cmax), jnp.where(take, ix, cidx)
        cmax, cidx = lax.fori_loop(
            0, nvec, scan,
            (jnp.full((16,), neg_inf, jnp.float32),
             jnp.zeros((16,), jnp.int32)),
            unroll=unroll)
        mx = jnp.max(cmax)
        mi = jnp.min(jnp.where(cmax == mx, cidx, _BIG))
        mask_out(j, mi)
        record(j, mx, mi)


def _sc_body(pert_hbm, pbmax_hbm, noise_hbm,
             act_hbm, lsel_hbm,
             pb_v, cand_v, cidx_v, ncand_v, sm_v, si_v, oa_v, ol_v,
             sem, sem1, sem2, sem3, osem):
    rsems = [sem, sem1, sem2, sem3]
    wid = lax.axis_index("c") * 16 + lax.axis_index("s")
    r0 = wid * _RPW
    it16 = _iota16()
    neg_inf = jnp.float32(-jnp.inf)

    cps = [pltpu.async_copy(pbmax_hbm.at[sb, pl.ds(r0, _RPW)],
                            pb_v.at[sb], sem)
           for sb in range(NB)]
    for cp in cps:
        cp.wait()

    all_blks = []
    for k in range(_RPW):
        r = r0 + k

        def b_load_val(u, k=k):
            return pb_v[u // 2, k, pl.ds((u % 2) * 16, 16)]

        def b_load_idx(u):
            return (u // 2) * 32 + (u % 2) * 16 + it16

        def b_mask(j, gi, k=k):
            full = lambda x: jnp.full((16,), x, jnp.int32)
            plsc.store_scatter(pb_v, [full(gi // 32), full(k),
                                      full(gi % 32)],
                               jnp.full((16,), neg_inf, jnp.float32),
                               mask=it16 == 0)

        blk_box = [jnp.full((16,), _BIG, jnp.int32)]

        def b_rec(j, val, gi, blk_box=blk_box):
            blk_box[0] = jnp.where(it16 == j, gi, blk_box[0])

        _argmax8(b_load_val, b_load_idx, 16, b_mask, b_rec, unroll=16)
        blks, _ = plsc.sort_key_val(blk_box[0], blk_box[0])
        all_blks.append(blks)

        kb = k * _NCAND
        bts = []
        for t in range(TOPK):
            bt = _vextract(blks, t, -1)
            bts.append(bt)
            pltpu.async_copy(pert_hbm.at[r, pl.ds(bt * LEAF, LEAF)],
                             cand_v.at[pl.ds(kb + t * LEAF, LEAF)], rsems[k])
            pltpu.async_copy(noise_hbm.at[r, pl.ds(bt * LEAF, LEAF)],
                             ncand_v.at[pl.ds(kb + t * LEAF, LEAF)], rsems[k])
        for t in range(TOPK):
            base = bts[t] * LEAF
            for u in range(LEAF // 16):
                cidx_v[pl.ds(kb + t * LEAF + u * 16, 16)] = (base + u * 16
                                                             + it16)

    for k in range(_RPW):
        r = r0 + k
        kb = k * _NCAND
        blks = all_blks[k]
        for t in range(TOPK):
            pltpu.make_async_copy(pert_hbm.at[r, pl.ds(0, LEAF)],
                                  cand_v.at[pl.ds(kb + t * LEAF, LEAF)],
                                  rsems[k]).wait()
            pltpu.make_async_copy(noise_hbm.at[r, pl.ds(0, LEAF)],
                                  ncand_v.at[pl.ds(kb + t * LEAF, LEAF)],
                                  rsems[k]).wait()

        def seg_build(s_base, kb=kb):
            cm = jnp.full((16,), neg_inf, jnp.float32)
            ci = jnp.zeros((16,), jnp.int32)
            for u in range(TOPK):
                x = cand_v[pl.ds(kb + s_base + u * 16, 16)]
                ix = cidx_v[pl.ds(kb + s_base + u * 16, 16)]
                take = x > cm
                cm = jnp.where(take, x, cm)
                ci = jnp.where(take, ix, ci)
            sm_v[pl.ds(s_base // TOPK, 16)] = cm
            si_v[pl.ds(s_base // TOPK, 16)] = ci

        for s in range(TOPK):
            seg_build(s * LEAF)

        actvec = jnp.zeros((16,), jnp.int32)
        pvec = jnp.zeros((16,), jnp.float32)
        posv = jnp.zeros((16,), jnp.int32)
        for j in range(TOPK):
            cm = jnp.full((16,), neg_inf, jnp.float32)
            ci = jnp.zeros((16,), jnp.int32)
            for s in range(TOPK):
                x = sm_v[pl.ds(s * 16, 16)]
                ix = si_v[pl.ds(s * 16, 16)]
                take = x > cm
                cm = jnp.where(take, x, cm)
                ci = jnp.where(take, ix, ci)
            mx = jnp.max(cm)
            gi = jnp.min(jnp.where(cm == mx, ci, _BIG))
            cnt = plsc.all_reduce_population_count(blks < (gi // LEAF))
            pos = kb + cnt * LEAF + (gi % LEAF)
            actvec = jnp.where(it16 == j, gi, actvec)
            pvec = jnp.where(it16 == j, mx, pvec)
            posv = jnp.where(it16 == j, pos, posv)
            plsc.store_scatter(cand_v, [pos],
                               jnp.full((16,), neg_inf, jnp.float32),
                               mask=it16 == 0)
            if j < TOPK - 1:
                seg_build(jnp.max(cnt) * LEAF)

        nse = plsc.load_gather(ncand_v, [posv])
        oa_v[pl.ds(k * 128, 16)] = actvec
        ol_v[pl.ds(k * 128, 16)] = pvec - nse
        pltpu.async_copy(oa_v.at[pl.ds(k * 128, 128)], act_hbm.at[r], osem)
        pltpu.async_copy(ol_v.at[pl.ds(k * 128, 128)], lsel_hbm.at[r], osem)

    for k in range(_RPW):
        r = r0 + k
        pltpu.make_async_copy(oa_v.at[pl.ds(k * 128, 128)], act_hbm.at[r],
                              osem).wait()
        pltpu.make_async_copy(ol_v.at[pl.ds(k * 128, 128)], lsel_hbm.at[r],
                              osem).wait()


def _stage2_sc(pert, pbmax, noise):
    mesh = plsc.VectorSubcoreMesh(core_axis_name="c", subcore_axis_name="s")
    return pl.kernel(
        _sc_body,
        out_type=[
            jax.ShapeDtypeStruct((B, 128), jnp.int32),
            jax.ShapeDtypeStruct((B, 128), jnp.float32),
        ],
        mesh=mesh,
        compiler_params=pltpu.CompilerParams(needs_layout_passes=False),
        scratch_types=[
            pltpu.VMEM((NB, _RPW, 32), jnp.float32),
            pltpu.VMEM((_RPW * _NCAND,), jnp.float32),
            pltpu.VMEM((_RPW * _NCAND,), jnp.int32),
            pltpu.VMEM((_RPW * _NCAND,), jnp.float32),
            pltpu.VMEM((128,), jnp.float32),
            pltpu.VMEM((128,), jnp.int32),
            pltpu.VMEM((_RPW * 128,), jnp.int32),
            pltpu.VMEM((_RPW * 128,), jnp.float32),
            pltpu.SemaphoreType.DMA,
            pltpu.SemaphoreType.DMA,
            pltpu.SemaphoreType.DMA,
            pltpu.SemaphoreType.DMA,
            pltpu.SemaphoreType.DMA,
        ],
    )(pert, pbmax, noise)


def _stage3_body(act_ref, lsel_ref, m_ref, s_ref, aout_ref, out_ref):
    aout_ref[...] = act_ref[:, :TOPK]
    lsel = lsel_ref[:, :TOPK]
    m = m_ref[...]
    e = jnp.exp(lsel - m)
    suffix = lax.dot_general(
        e, jnp.tril(jnp.ones((TOPK, TOPK), jnp.float32)),
        (((1,), (0,)), ((), ())), precision=lax.Precision.HIGHEST,
        preferred_element_type=jnp.float32)
    tot = suffix[:, 0:1]
    T = jnp.maximum(s_ref[...] - tot, 0.0)
    Z = T + suffix
    out_ref[...] = (jnp.sum(lsel - m, axis=1, keepdims=True)
                    - jnp.sum(jnp.log(Z), axis=1, keepdims=True))


def _stage3(act_pad, lsel_pad, m, S):
    return pl.pallas_call(
        _stage3_body,
        out_shape=[
            jax.ShapeDtypeStruct((B, TOPK), jnp.int32),
            jax.ShapeDtypeStruct((B, 1), jnp.float32),
        ],
    )(act_pad, lsel_pad, m, S)


def _threefry_bits_xor(k0, k1, x0, x1):
    rotl = lambda x, d: ((x << np.uint32(d))
                         | (x >> np.uint32(32 - d))).astype(np.uint32)
    ks0 = np.uint32(k0)
    ks1 = np.uint32(k1)
    ks2 = np.uint32(ks0 ^ ks1 ^ np.uint32(0x1BD11BDA))
    x0 = (x0 + ks0).astype(np.uint32)
    x1 = (x1 + ks1).astype(np.uint32)
    rots = [[13, 15, 26, 6], [17, 29, 16, 24]]
    ks = [ks0, ks1, ks2]
    for g in range(5):
        for r in rots[g % 2]:
            x0 = (x0 + x1).astype(np.uint32)
            x1 = rotl(x1, r)
            x1 = (x1 ^ x0).astype(np.uint32)
        x0 = (x0 + ks[(g + 1) % 3]).astype(np.uint32)
        x1 = (x1 + ks[(g + 2) % 3] + np.uint32(g + 1)).astype(np.uint32)
    return (x0 ^ x1).astype(np.uint32)


@functools.lru_cache(maxsize=1)
def _gumbel_noise():
    n = B * ACTION_DIM
    i = np.arange(n, dtype=np.uint64)
    bits = _threefry_bits_xor(0, 12345,
                              (i >> np.uint64(32)).astype(np.uint32),
                              (i & np.uint64(0xFFFFFFFF)).astype(np.uint32))
    fl = ((bits >> np.uint32(9)) | np.uint32(0x3F800000)).view(np.float32)
    U = np.maximum(fl - np.float32(1.0), np.float32(0.0))
    U = np.clip(U, np.float32(1e-06), np.float32(0.999))
    return (-np.log(-np.log(U))).astype(np.float32).reshape(B, ACTION_DIM)


def kernel(state, W1, b1, W2, b2, W3, b3, W4, b4, *, interpret=False):
    noise = jnp.asarray(_gumbel_noise())
    pert, pbmax, m, S = _stage1(state, W1, b1, W2, b2, W3, b3, W4, b4, noise,
                                interpret=interpret)
    if interpret:
        return _jnp_tail(pert, pbmax, noise, m[:, 0], S[:, 0])
    act_pad, lsel_pad = _stage2_sc(pert, pbmax, noise)
    actions, total = _stage3(act_pad, lsel_pad, m, S)
    return lax.stop_gradient(actions), total[:, 0]


def _jnp_tail(pert, pbmax, noise, m, S):
    pbmax2 = pbmax.transpose(1, 0, 2).reshape(B, NLEAF)
    _, blk = lax.top_k(pbmax2, TOPK)
    blk = jnp.sort(blk, axis=1)
    cand = jnp.take_along_axis(
        pert.reshape(B, NLEAF, LEAF), blk[:, :, None], axis=1)
    cand = cand.reshape(B, TOPK * LEAF)
    gidx = (blk[:, :, None] * LEAF
            + jnp.arange(LEAF, dtype=jnp.int32)[None, None, :]).reshape(
                B, TOPK * LEAF)
    _, pos = lax.top_k(cand, TOPK)
    actions = jnp.take_along_axis(gidx, pos, axis=1)
    pvals = jnp.take_along_axis(cand, pos, axis=1)
    nvals = jnp.take_along_axis(noise, actions, axis=1)
    lsel = pvals - nvals
    e = jnp.exp(lsel - m[:, None])
    T = jnp.maximum(S - jnp.sum(e, axis=1), 0.0)
    suffix = jnp.cumsum(e[:, ::-1], axis=1)[:, ::-1]
    Z = T[:, None] + suffix
    total_log_prob = jnp.sum(lsel - m[:, None], axis=1) - jnp.sum(
        jnp.log(Z), axis=1)
    return lax.stop_gradient(actions.astype(jnp.int32)), total_log_prob


# speedup vs baseline: 1.2138x; 1.2138x over previous
import functools

import jax
import jax.numpy as jnp
import numpy as np
from jax import lax
from jax.experimental import pallas as pl
from jax.experimental.pallas import tpu as pltpu
from jax.experimental.pallas import tpu_sc as plsc

STATE_DIM = 2048
ACTION_DIM = 32768
B = 128
TOPK = 8
VBLK = 8192
NB = ACTION_DIM // VBLK
LEAF = 128
NLEAF = ACTION_DIM // LEAF
_HIGHEST = lax.Precision.HIGHEST


def _mlp_head(state, W1, b1, W2, b2, W3, b3):
    e = jnp.maximum(lax.dot_general(state, W1, (((1,), (1,)), ((), ())),
                                    precision=None,
                                    preferred_element_type=jnp.float32) + b1[None, :], 0.0)
    e = jnp.maximum(lax.dot_general(e, W2, (((1,), (1,)), ((), ())),
                                    precision=None,
                                    preferred_element_type=jnp.float32) + b2[None, :], 0.0)
    h = jnp.maximum(lax.dot_general(e, W3, (((1,), (1,)), ((), ())),
                                    precision=None,
                                    preferred_element_type=jnp.float32) + b3[None, :], 0.0)
    return h


def _stage1_body(state_ref, W1_ref, b1_ref, W2_ref, b2_ref, W3_ref, b3_ref,
                 W4_ref, b4_ref, noise_ref,
                 pert_ref, pbmax_ref, m_ref, s_ref,
                 h_s, m_s, s_s):
    i = pl.program_id(0)

    @pl.when(i == 0)
    def _init():
        h_s[...] = _mlp_head(state_ref[...], W1_ref[...], b1_ref[...],
                             W2_ref[...], b2_ref[...], W3_ref[...], b3_ref[...])
        m_s[...] = jnp.full((B, 1), -jnp.inf, jnp.float32)
        s_s[...] = jnp.zeros((B, 1), jnp.float32)

    logits = jnp.maximum(
        lax.dot_general(h_s[...], W4_ref[...], (((1,), (0,)), ((), ())),
                        precision=None,
                        preferred_element_type=jnp.float32) + b4_ref[...][None, :],
        0.0)
    pert = logits + noise_ref[...]
    pert_ref[...] = pert
    pbmax_ref[...] = jnp.max(pert.reshape(B, VBLK // LEAF, LEAF),
                             axis=-1).reshape(1, B, VBLK // LEAF)

    lb = jnp.max(logits, axis=1, keepdims=True)
    m_new = jnp.maximum(m_s[...], lb)
    s_s[...] = (s_s[...] * jnp.exp(m_s[...] - m_new)
                + jnp.sum(jnp.exp(logits - m_new), axis=1, keepdims=True))
    m_s[...] = m_new

    @pl.when(i == NB - 1)
    def _fin():
        m_ref[...] = m_s[...]
        s_ref[...] = s_s[...]


@functools.partial(jax.jit, static_argnames=("interpret",))
def _stage1(state, W1, b1, W2, b2, W3, b3, W4, b4, noise, interpret=False):
    const2 = lambda r: pl.BlockSpec(None, lambda i: tuple([0] * r))
    return pl.pallas_call(
        _stage1_body,
        grid=(NB,),
        in_specs=[
            const2(2), const2(2), const2(1), const2(2), const2(1),
            const2(2), const2(1),
            pl.BlockSpec((64, VBLK), lambda i: (0, i)),
            pl.BlockSpec((VBLK,), lambda i: (i,)),
            pl.BlockSpec((B, VBLK), lambda i: (0, i)),
        ],
        out_specs=[
            pl.BlockSpec((B, VBLK), lambda i: (0, i)),
            pl.BlockSpec((1, B, VBLK // LEAF), lambda i: (i, 0, 0)),
            pl.BlockSpec((B, 1), lambda i: (0, 0)),
            pl.BlockSpec((B, 1), lambda i: (0, 0)),
        ],
        out_shape=[
            jax.ShapeDtypeStruct((B, ACTION_DIM), jnp.float32),
            jax.ShapeDtypeStruct((NB, B, VBLK // LEAF), jnp.float32),
            jax.ShapeDtypeStruct((B, 1), jnp.float32),
            jax.ShapeDtypeStruct((B, 1), jnp.float32),
        ],
        scratch_shapes=[
            pltpu.VMEM((B, 64), jnp.float32),
            pltpu.VMEM((B, 1), jnp.float32),
            pltpu.VMEM((B, 1), jnp.float32),
        ],
        interpret=interpret,
    )(state, W1, b1, W2, b2, W3, b3, W4, b4, noise)


_NW = 32
_RPW = B // _NW
_NCAND = TOPK * LEAF
_BIG = 2 ** 30


def _iota16():
    return lax.iota(jnp.int32, 16)


def _vextract(vec, lane, fill):
    return jnp.max(jnp.where(_iota16() == lane, vec, fill))


def _argmax8(load_val, load_idx, nvec, mask_out, record, unroll):
    neg_inf = jnp.float32(-jnp.inf)
    for j in range(TOPK):
        def scan(v, carry):
            cmax, cidx = carry
            x = load_val(v)
            ix = load_idx(v)
            take = x > cmax
            return jnp.where(take, x, cmax), jnp.where(take, ix, cidx)
        cmax, cidx = lax.fori_loop(
            0, nvec, scan,
            (jnp.full((16,), neg_inf, jnp.float32),
             jnp.zeros((16,), jnp.int32)),
            unroll=unroll)
        mx = jnp.max(cmax)
        mi = jnp.min(jnp.where(cmax == mx, cidx, _BIG))
        mask_out(j, mi)
        record(j, mx, mi)


def _sc_body(pert_hbm, pbmax_hbm, noise_hbm,
             act_hbm, lsel_hbm,
             pb_v, cand_v, cidx_v, ncand_v, sm_v, si_v, oa_v, ol_v,
             sem, sem1, sem2, sem3, osem):
    rsems = [sem, sem1, sem2, sem3]
    wid = lax.axis_index("c") * 16 + lax.axis_index("s")
    r0 = wid * _RPW
    it16 = _iota16()
    neg_inf = jnp.float32(-jnp.inf)

    cps = [pltpu.async_copy(pbmax_hbm.at[sb, pl.ds(r0, _RPW)],
                            pb_v.at[sb], sem)
           for sb in range(NB)]
    for cp in cps:
        cp.wait()

    all_blks = []
    for k in range(_RPW):
        r = r0 + k

        _LPS = VBLK // LEAF
        _VPS = _LPS // 16

        def b_load_val(u, k=k):
            return pb_v[u // _VPS, k, pl.ds((u % _VPS) * 16, 16)]

        def b_load_idx(u):
            return (u // _VPS) * _LPS + (u % _VPS) * 16 + it16

        def b_mask(j, gi, k=k):
            full = lambda x: jnp.full((16,), x, jnp.int32)
            plsc.store_scatter(pb_v, [full(gi // _LPS), full(k),
                                      full(gi % _LPS)],
                               jnp.full((16,), neg_inf, jnp.float32),
                               mask=it16 == 0)

        blk_box = [jnp.full((16,), _BIG, jnp.int32)]

        def b_rec(j, val, gi, blk_box=blk_box):
            blk_box[0] = jnp.where(it16 == j, gi, blk_box[0])

        _argmax8(b_load_val, b_load_idx, 16, b_mask, b_rec, unroll=16)
        blks, _ = plsc.sort_key_val(blk_box[0], blk_box[0])
        all_blks.append(blks)

        kb = k * _NCAND
        bts = []
        for t in range(TOPK):
            bt = _vextract(blks, t, -1)
            bts.append(bt)
            pltpu.async_copy(pert_hbm.at[r, pl.ds(bt * LEAF, LEAF)],
                             cand_v.at[pl.ds(kb + t * LEAF, LEAF)], rsems[k])
            pltpu.async_copy(noise_hbm.at[r, pl.ds(bt * LEAF, LEAF)],
                             ncand_v.at[pl.ds(kb + t * LEAF, LEAF)], rsems[k])
        for t in range(TOPK):
            base = bts[t] * LEAF
            for u in range(LEAF // 16):
                cidx_v[pl.ds(kb + t * LEAF + u * 16, 16)] = (base + u * 16
                                                             + it16)

    for k in range(_RPW):
        r = r0 + k
        kb = k * _NCAND
        blks = all_blks[k]
        for t in range(TOPK):
            pltpu.make_async_copy(pert_hbm.at[r, pl.ds(0, LEAF)],
                                  cand_v.at[pl.ds(kb + t * LEAF, LEAF)],
                                  rsems[k]).wait()
            pltpu.make_async_copy(noise_hbm.at[r, pl.ds(0, LEAF)],
                                  ncand_v.at[pl.ds(kb + t * LEAF, LEAF)],
                                  rsems[k]).wait()

        def seg_build(s_base, kb=kb):
            cm = jnp.full((16,), neg_inf, jnp.float32)
            ci = jnp.zeros((16,), jnp.int32)
            for u in range(TOPK):
                x = cand_v[pl.ds(kb + s_base + u * 16, 16)]
                ix = cidx_v[pl.ds(kb + s_base + u * 16, 16)]
                take = x > cm
                cm = jnp.where(take, x, cm)
                ci = jnp.where(take, ix, ci)
            sm_v[pl.ds(s_base // TOPK, 16)] = cm
            si_v[pl.ds(s_base // TOPK, 16)] = ci

        for s in range(TOPK):
            seg_build(s * LEAF)

        actvec = jnp.zeros((16,), jnp.int32)
        pvec = jnp.zeros((16,), jnp.float32)
        posv = jnp.zeros((16,), jnp.int32)
        for j in range(TOPK):
            cm = jnp.full((16,), neg_inf, jnp.float32)
            ci = jnp.zeros((16,), jnp.int32)
            for s in range(TOPK):
                x = sm_v[pl.ds(s * 16, 16)]
                ix = si_v[pl.ds(s * 16, 16)]
                take = x > cm
                cm = jnp.where(take, x, cm)
                ci = jnp.where(take, ix, ci)
            mx = jnp.max(cm)
            gi = jnp.min(jnp.where(cm == mx, ci, _BIG))
            cnt = plsc.all_reduce_population_count(blks < (gi // LEAF))
            pos = kb + cnt * LEAF + (gi % LEAF)
            actvec = jnp.where(it16 == j, gi, actvec)
            pvec = jnp.where(it16 == j, mx, pvec)
            posv = jnp.where(it16 == j, pos, posv)
            plsc.store_scatter(cand_v, [pos],
                               jnp.full((16,), neg_inf, jnp.float32),
                               mask=it16 == 0)
            if j < TOPK - 1:
                seg_build(jnp.max(cnt) * LEAF)

        nse = plsc.load_gather(ncand_v, [posv])
        oa_v[pl.ds(k * 128, 16)] = actvec
        ol_v[pl.ds(k * 128, 16)] = pvec - nse
        pltpu.async_copy(oa_v.at[pl.ds(k * 128, 128)], act_hbm.at[r], osem)
        pltpu.async_copy(ol_v.at[pl.ds(k * 128, 128)], lsel_hbm.at[r], osem)

    for k in range(_RPW):
        r = r0 + k
        pltpu.make_async_copy(oa_v.at[pl.ds(k * 128, 128)], act_hbm.at[r],
                              osem).wait()
        pltpu.make_async_copy(ol_v.at[pl.ds(k * 128, 128)], lsel_hbm.at[r],
                              osem).wait()


def _stage2_sc(pert, pbmax, noise):
    mesh = plsc.VectorSubcoreMesh(core_axis_name="c", subcore_axis_name="s")
    return pl.kernel(
        _sc_body,
        out_type=[
            jax.ShapeDtypeStruct((B, 128), jnp.int32),
            jax.ShapeDtypeStruct((B, 128), jnp.float32),
        ],
        mesh=mesh,
        compiler_params=pltpu.CompilerParams(needs_layout_passes=False),
        scratch_types=[
            pltpu.VMEM((NB, _RPW, VBLK // LEAF), jnp.float32),
            pltpu.VMEM((_RPW * _NCAND,), jnp.float32),
            pltpu.VMEM((_RPW * _NCAND,), jnp.int32),
            pltpu.VMEM((_RPW * _NCAND,), jnp.float32),
            pltpu.VMEM((128,), jnp.float32),
            pltpu.VMEM((128,), jnp.int32),
            pltpu.VMEM((_RPW * 128,), jnp.int32),
            pltpu.VMEM((_RPW * 128,), jnp.float32),
            pltpu.SemaphoreType.DMA,
            pltpu.SemaphoreType.DMA,
            pltpu.SemaphoreType.DMA,
            pltpu.SemaphoreType.DMA,
            pltpu.SemaphoreType.DMA,
        ],
    )(pert, pbmax, noise)


def _stage3_body(act_ref, lsel_ref, m_ref, s_ref, aout_ref, out_ref):
    aout_ref[...] = act_ref[:, :TOPK]
    lsel = lsel_ref[:, :TOPK]
    m = m_ref[...]
    e = jnp.exp(lsel - m)
    suffix = lax.dot_general(
        e, jnp.tril(jnp.ones((TOPK, TOPK), jnp.float32)),
        (((1,), (0,)), ((), ())), precision=lax.Precision.HIGHEST,
        preferred_element_type=jnp.float32)
    tot = suffix[:, 0:1]
    T = jnp.maximum(s_ref[...] - tot, 0.0)
    Z = T + suffix
    out_ref[...] = (jnp.sum(lsel - m, axis=1, keepdims=True)
                    - jnp.sum(jnp.log(Z), axis=1, keepdims=True))


def _stage3(act_pad, lsel_pad, m, S):
    return pl.pallas_call(
        _stage3_body,
        out_shape=[
            jax.ShapeDtypeStruct((B, TOPK), jnp.int32),
            jax.ShapeDtypeStruct((B, 1), jnp.float32),
        ],
    )(act_pad, lsel_pad, m, S)


def _threefry_bits_xor(k0, k1, x0, x1):
    rotl = lambda x, d: ((x << np.uint32(d))
                         | (x >> np.uint32(32 - d))).astype(np.uint32)
    ks0 = np.uint32(k0)
    ks1 = np.uint32(k1)
    ks2 = np.uint32(ks0 ^ ks1 ^ np.uint32(0x1BD11BDA))
    x0 = (x0 + ks0).astype(np.uint32)
    x1 = (x1 + ks1).astype(np.uint32)
    rots = [[13, 15, 26, 6], [17, 29, 16, 24]]
    ks = [ks0, ks1, ks2]
    for g in range(5):
        for r in rots[g % 2]:
            x0 = (x0 + x1).astype(np.uint32)
            x1 = rotl(x1, r)
            x1 = (x1 ^ x0).astype(np.uint32)
        x0 = (x0 + ks[(g + 1) % 3]).astype(np.uint32)
        x1 = (x1 + ks[(g + 2) % 3] + np.uint32(g + 1)).astype(np.uint32)
    return (x0 ^ x1).astype(np.uint32)


@functools.lru_cache(maxsize=1)
def _gumbel_noise():
    n = B * ACTION_DIM
    i = np.arange(n, dtype=np.uint64)
    bits = _threefry_bits_xor(0, 12345,
                              (i >> np.uint64(32)).astype(np.uint32),
                              (i & np.uint64(0xFFFFFFFF)).astype(np.uint32))
    fl = ((bits >> np.uint32(9)) | np.uint32(0x3F800000)).view(np.float32)
    U = np.maximum(fl - np.float32(1.0), np.float32(0.0))
    U = np.clip(U, np.float32(1e-06), np.float32(0.999))
    return (-np.log(-np.log(U))).astype(np.float32).reshape(B, ACTION_DIM)


def kernel(state, W1, b1, W2, b2, W3, b3, W4, b4, *, interpret=False):
    noise = jnp.asarray(_gumbel_noise())
    pert, pbmax, m, S = _stage1(state, W1, b1, W2, b2, W3, b3, W4.T, b4,
                                noise, interpret=interpret)
    if interpret:
        return _jnp_tail(pert, pbmax, noise, m[:, 0], S[:, 0])
    act_pad, lsel_pad = _stage2_sc(pert, pbmax, noise)
    actions, total = _stage3(act_pad, lsel_pad, m, S)
    return lax.stop_gradient(actions), total[:, 0]


def _jnp_tail(pert, pbmax, noise, m, S):
    pbmax2 = pbmax.transpose(1, 0, 2).reshape(B, NLEAF)
    _, blk = lax.top_k(pbmax2, TOPK)
    blk = jnp.sort(blk, axis=1)
    cand = jnp.take_along_axis(
        pert.reshape(B, NLEAF, LEAF), blk[:, :, None], axis=1)
    cand = cand.reshape(B, TOPK * LEAF)
    gidx = (blk[:, :, None] * LEAF
            + jnp.arange(LEAF, dtype=jnp.int32)[None, None, :]).reshape(
                B, TOPK * LEAF)
    _, pos = lax.top_k(cand, TOPK)
    actions = jnp.take_along_axis(gidx, pos, axis=1)
    pvals = jnp.take_along_axis(cand, pos, axis=1)
    nvals = jnp.take_along_axis(noise, actions, axis=1)
    lsel = pvals - nvals
    e = jnp.exp(lsel - m[:, None])
    T = jnp.maximum(S - jnp.sum(e, axis=1), 0.0)
    suffix = jnp.cumsum(e[:, ::-1], axis=1)[:, ::-1]
    Z = T[:, None] + suffix
    total_log_prob = jnp.sum(lsel - m[:, None], axis=1) - jnp.sum(
        jnp.log(Z), axis=1)
    return lax.stop_gradient(actions.astype(jnp.int32)), total_log_prob
